# baseline (speedup 1.0000x reference)
import functools

import jax
import jax.numpy as jnp
from jax import lax
from jax.experimental import pallas as pl
from jax.experimental.pallas import tpu as pltpu
from jax.experimental.pallas import tpu_sc as plsc

D_MODEL = 64
SCALE = 0.125
NBUF = 4
C = 128
R1 = 160


@jax.jit
def _embed(x, table):
    idx = x.reshape(-1)
    B = idx.shape[0]
    V = table.shape[0]

    info = plsc.get_sparse_core_info()
    NC, NS = info.num_cores, info.num_subcores
    NW = NC * NS
    b_per_w = B // NW
    assert b_per_w * NW == B
    n_chunks = b_per_w // C
    assert n_chunks * C == b_per_w
    assert n_chunks >= 6 and (n_chunks - 4) % NBUF == 0
    n_outer = (n_chunks - 4) // NBUF

    k1_chunks = V // R1
    assert k1_chunks * R1 == V
    k1_iters = (k1_chunks + NW - 1) // NW

    mesh = plsc.VectorSubcoreMesh(core_axis_name="c", subcore_axis_name="s")

    @functools.partial(
        pl.kernel,
        mesh=mesh,
        compiler_params=pltpu.CompilerParams(use_tc_tiling_on_sc=True),
        out_type=jax.ShapeDtypeStruct((V, 2 * D_MODEL), jnp.float32),
        scratch_types=(
            [pltpu.VMEM((R1, D_MODEL), jnp.float32) for _ in range(2)]
            + [pltpu.VMEM((R1, 2 * D_MODEL), jnp.float32) for _ in range(2)]
            + [pltpu.SemaphoreType.DMA for _ in range(4)]
        ),
    )
    def k1(table_hbm, dup_hbm, vin0, vin1, vout0, vout1, gs0, gs1, ws0, ws1):
        vin = (vin0, vin1)
        vout = (vout0, vout1)
        gsem = (gs0, gs1)
        wsem = (ws0, ws1)
        wid = lax.axis_index("s") * NC + lax.axis_index("c")

        def rd_desc(g, b):
            return pltpu.make_async_copy(
                table_hbm.at[pl.ds(g * R1, R1)], vin[b], gsem[b])

        def wr_desc(g, b):
            return pltpu.make_async_copy(
                vout[b], dup_hbm.at[pl.ds(g * R1, R1)], wsem[b])

        def scale_dup(b):
            def sbody(i, c):
                for j in range(D_MODEL // 16):
                    v = vin[b][i, pl.ds(j * 16, 16)] * SCALE
                    vout[b][i, pl.ds(j * 16, 16)] = v
                    vout[b][i, pl.ds(D_MODEL + j * 16, 16)] = v
                return c

            lax.fori_loop(0, R1, sbody, 0)

        def chunk_of(j):
            return j * NW + wid

        def body(j, c):
            @pl.when(chunk_of(j) < k1_chunks)
            def _():
                rd_desc(chunk_of(j), 0).start()
                rd_desc(chunk_of(j), 0).wait()
                scale_dup(0)
                wr_desc(chunk_of(j), 0).start()
                wr_desc(chunk_of(j), 0).wait()

            return c

        lax.fori_loop(0, k1_iters, body, 0)

    @functools.partial(
        pl.kernel,
        mesh=mesh,
        compiler_params=pltpu.CompilerParams(use_tc_tiling_on_sc=True),
        out_type=jax.ShapeDtypeStruct((B, 2 * D_MODEL), jnp.float32),
        scratch_types=(
            [pltpu.VMEM((b_per_w,), jnp.int32)]
            + [pltpu.VMEM((C, 2 * D_MODEL), jnp.float32) for _ in range(NBUF)]
            + [pltpu.SemaphoreType.DMA for _ in range(2 * NBUF)]
        ),
    )
    def k2(dup_hbm, idx_hbm, out_hbm, idx_all, *bufs):
        rows = bufs[:NBUF]
        gsem = bufs[NBUF:2 * NBUF]
        wsem = bufs[2 * NBUF:]
        wid = lax.axis_index("s") * NC + lax.axis_index("c")
        base = wid * b_per_w

        def gather_desc(g, b):
            isl = idx_all.at[pl.ds(g * C, C)]
            return pltpu.make_async_copy(dup_hbm.at[isl], rows[b], gsem[b])

        def wb_desc(g, b):
            return pltpu.make_async_copy(
                rows[b], out_hbm.at[pl.ds(base + g * C, C)], wsem[b])

        pltpu.sync_copy(idx_hbm.at[pl.ds(base, b_per_w)], idx_all)

        gather_desc(0, 0).start()
        gather_desc(1, 1).start()

        for g in (0, 1):
            b = g % NBUF
            gather_desc(g, b).wait()
            wb_desc(g, b).start()
            gather_desc(g + 2, (g + 2) % NBUF).start()

        def outer(go, c):
            for k in range(NBUF):
                g = 2 + go * NBUF + k
                b = (2 + k) % NBUF
                b2 = k
                gather_desc(g, b).wait()
                wb_desc(g, b).start()
                wb_desc(g - 2, b2).wait()
                gather_desc(g + 2, b2).start()
            return c

        lax.fori_loop(0, n_outer, outer, 0)

        for g in (n_chunks - 2, n_chunks - 1):
            b = g % NBUF
            gather_desc(g, b).wait()
            wb_desc(g, b).start()

        for g in range(n_chunks - NBUF, n_chunks):
            wb_desc(g, g % NBUF).wait()

    dup = k1(table)
    out = k2(dup, idx)
    return out[:, :D_MODEL].reshape(x.shape + (D_MODEL,))


def kernel(x, table):
    return _embed(x, table)
